# Initial kernel scaffold; baseline (speedup 1.0000x reference)
import jax
import jax.numpy as jnp
from jax.experimental import pallas as pl


def kernel(input, table):
    raise NotImplementedError("write your pallas kernel here")


# speedup vs baseline: 3.4259x; 3.4259x over previous
import functools

import jax
import jax.numpy as jnp
from jax import lax
from jax.experimental import pallas as pl
from jax.experimental.pallas import tpu as pltpu
from jax.experimental.pallas import tpu_sc as plsc

VOCAB = 1002
N_D = 64
BATCH = 4096
HIST = 200

B = BATCH * HIST
NW = 32
CH = 128
BPW = B // NW
NCH = BPW // CH

_mesh = plsc.VectorSubcoreMesh(core_axis_name="c", subcore_axis_name="s")


@functools.partial(
    pl.kernel,
    mesh=_mesh,
    out_type=jax.ShapeDtypeStruct((B, N_D), jnp.float32),
    scratch_types=[
        pltpu.VMEM((NCH, CH), jnp.int32),
        pltpu.VMEM((CH, N_D), jnp.float32),
        pltpu.SemaphoreType.DMA,
    ],
    compiler_params=pltpu.CompilerParams(use_tc_tiling_on_sc=False),
)
def _sc_embed(idx_hbm, table_hbm, out_hbm, idx_v, rows_v, gsem):
    c = lax.axis_index("c")
    s = lax.axis_index("s")
    wid = s * 2 + c
    pltpu.sync_copy(idx_hbm.at[pl.ds(wid * NCH, NCH)], idx_v)
    out_base = wid * BPW

    def body(j, carry):
        pltpu.async_copy(table_hbm.at[idx_v.at[j]], rows_v, gsem).wait()
        pltpu.sync_copy(rows_v, out_hbm.at[pl.ds(out_base + j * CH, CH)])
        return carry

    lax.fori_loop(0, NCH, body, 0)


def kernel(input, table):
    idx = input.reshape(B // CH, CH).astype(jnp.int32)
    out = _sc_embed(idx, table)
    return out.reshape(BATCH, HIST, N_D)


# speedup vs baseline: 3.5769x; 1.0440x over previous
import functools

import jax
import jax.numpy as jnp
from jax import lax
from jax.experimental import pallas as pl
from jax.experimental.pallas import tpu as pltpu
from jax.experimental.pallas import tpu_sc as plsc

VOCAB = 1002
N_D = 64
BATCH = 4096
HIST = 200

B = BATCH * HIST
NW = 32
CH = 128
BPW = B // NW
NCH = BPW // CH

_mesh = plsc.VectorSubcoreMesh(core_axis_name="c", subcore_axis_name="s")

NBUF = 4
NG = NCH // NBUF


@functools.partial(
    pl.kernel,
    mesh=_mesh,
    out_type=jax.ShapeDtypeStruct((B, N_D), jnp.float32),
    scratch_types=[
        pltpu.VMEM((NCH, CH), jnp.int32),
        pltpu.VMEM((NBUF, CH, N_D), jnp.float32),
        pltpu.SemaphoreType.DMA((NBUF,)),
        pltpu.SemaphoreType.DMA((NBUF,)),
    ],
    compiler_params=pltpu.CompilerParams(use_tc_tiling_on_sc=False),
)
def _sc_embed(idx_hbm, table_hbm, out_hbm, idx_v, rows_v, gsem, ssem):
    c = lax.axis_index("c")
    s = lax.axis_index("s")
    wid = s * 2 + c
    pltpu.sync_copy(idx_hbm.at[pl.ds(wid * NCH, NCH)], idx_v)
    out_base = wid * BPW

    def gather(j, b):
        return pltpu.async_copy(table_hbm.at[idx_v.at[j]], rows_v.at[b],
                                gsem.at[b])

    def wait_gather(j, b):
        pltpu.make_async_copy(table_hbm.at[idx_v.at[j]], rows_v.at[b],
                              gsem.at[b]).wait()

    def store(j, b):
        return pltpu.async_copy(rows_v.at[b],
                                out_hbm.at[pl.ds(out_base + j * CH, CH)],
                                ssem.at[b])

    def wait_store(j, b):
        pltpu.make_async_copy(rows_v.at[b],
                              out_hbm.at[pl.ds(out_base + j * CH, CH)],
                              ssem.at[b]).wait()

    for b in range(NBUF):
        gather(b, b)

    def group(g, carry):
        base = g * NBUF
        for b in range(NBUF):
            wait_gather(base + b, b)
            store(base + b, b)
        for b in range(NBUF):
            wait_store(base + b, b)
            gather(base + NBUF + b, b)
        return carry

    lax.fori_loop(0, NG - 1, group, 0, unroll=False)

    last = (NG - 1) * NBUF
    for b in range(NBUF):
        wait_gather(last + b, b)
        store(last + b, b)
    for b in range(NBUF):
        wait_store(last + b, b)


def kernel(input, table):
    idx = input.reshape(B // CH, CH).astype(jnp.int32)
    out = _sc_embed(idx, table)
    return out.reshape(BATCH, HIST, N_D)
